# baseline (speedup 1.0000x reference)
import jax
import jax.numpy as jnp
from jax import lax
from jax.experimental import pallas as pl
from jax.experimental.pallas import tpu as pltpu

B, C, H, W = 8, 768, 64, 64
K = 16
CC = C // K
AC = CC * 8
HW = H * W
LW = B * HW
CB = 64
NCB = C // CB
RB = 8
NRB = CC // RB
_NEG_INF = float("-inf")
_BF = jnp.bfloat16
_F32 = jnp.float32


def _reduce_body(x_ref, mx_ref, av_ref):
    xb = x_ref[...]
    mx_ref[...] = jnp.max(xb, axis=2)[None]
    av_ref[...] = (jnp.sum(xb, axis=2) * (1.0 / HW))[None]


def _reduce(x3):
    mx3, av3 = pl.pallas_call(
        _reduce_body,
        grid=(NCB,),
        in_specs=[pl.BlockSpec((B, CB, HW), lambda i: (0, i, 0))],
        out_specs=[pl.BlockSpec((1, B, CB), lambda i: (i, 0, 0)),
                   pl.BlockSpec((1, B, CB), lambda i: (i, 0, 0))],
        out_shape=[jax.ShapeDtypeStruct((NCB, B, CB), _F32),
                   jax.ShapeDtypeStruct((NCB, B, CB), _F32)],
    )(x3)
    to2d = lambda a: a.transpose(1, 0, 2).reshape(B, C)
    return to2d(mx3), to2d(av3)


def _se_topk_body(mx_ref, av_ref, w1_ref, w2_ref, num_ref, s_ref, idx_ref,
                  nsel_ref):
    v = jnp.concatenate([mx_ref[...], av_ref[...]], axis=0)
    h = jax.nn.relu(
        lax.dot_general(v, w1_ref[...], (((1,), (1,)), ((), ())),
                        preferred_element_type=_F32))
    r = lax.dot_general(h, w2_ref[...], (((1,), (1,)), ((), ())),
                        preferred_element_type=_F32)
    num = jax.nn.sigmoid(r[:B] + r[B:])
    num_ref[...] = num
    slist = jnp.sum(num, axis=0, keepdims=True)
    iota = lax.broadcasted_iota(jnp.int32, (1, C), 1)
    iota48 = lax.broadcasted_iota(jnp.int32, (1, CC), 1)

    def body(i, carry):
        cur, mask, idxv = carry
        m = jnp.max(cur)
        j = jnp.min(jnp.where(cur == m, iota, C))
        idxv = jnp.where(iota48 == i, j, idxv)
        hit = iota == j
        return (jnp.where(hit, _NEG_INF, cur),
                jnp.where(hit, 1.0, mask), idxv)

    _, mask, idxv = lax.fori_loop(
        0, CC, body,
        (slist, jnp.zeros((1, C), _F32), jnp.zeros((1, CC), jnp.int32)),
        unroll=False)
    idx_ref[...] = idxv
    s_ref[...] = jnp.where(mask > 0.5, 1.0, 1.0 + num)
    oh = (lax.broadcasted_iota(jnp.int32, (CC, C), 1)
          == idxv.reshape(CC, 1)).astype(_F32)
    nsel_ref[...] = lax.dot_general(oh, num, (((1,), (1,)), ((), ())),
                                    preferred_element_type=_F32)


def _se_topk(mx, av, ca_w1, ca_w2):
    return pl.pallas_call(
        _se_topk_body,
        in_specs=[
            pl.BlockSpec((B, C), lambda: (0, 0)),
            pl.BlockSpec((B, C), lambda: (0, 0)),
            pl.BlockSpec((C // 2, C), lambda: (0, 0)),
            pl.BlockSpec((C, C // 2), lambda: (0, 0)),
        ],
        out_specs=[pl.BlockSpec((B, C), lambda: (0, 0)),
                   pl.BlockSpec((B, C), lambda: (0, 0)),
                   pl.BlockSpec((1, CC), lambda: (0, 0)),
                   pl.BlockSpec((CC, B), lambda: (0, 0))],
        out_shape=[jax.ShapeDtypeStruct((B, C), _F32),
                   jax.ShapeDtypeStruct((B, C), _F32),
                   jax.ShapeDtypeStruct((1, CC), jnp.int32),
                   jax.ShapeDtypeStruct((CC, B), _F32)],
    )(mx, av, ca_w1, ca_w2)


def _gather_body(idx_ref, x_ref, out_ref):
    out_ref[...] = x_ref[...].reshape(1, B, H, W)


def _gather(idx, x):
    grid_spec = pltpu.PrefetchScalarGridSpec(
        num_scalar_prefetch=1,
        grid=(CC,),
        in_specs=[
            pl.BlockSpec((B, 1, H, W), lambda j, idx_ref: (0, idx_ref[j], 0, 0)),
        ],
        out_specs=pl.BlockSpec((1, B, H, W), lambda j, idx_ref: (j, 0, 0, 0)),
    )
    return pl.pallas_call(
        _gather_body,
        grid_spec=grid_spec,
        out_shape=jax.ShapeDtypeStruct((CC, B, H, W), _F32),
    )(idx, x)


def _geom():
    liota = lax.broadcasted_iota(jnp.int32, (1, LW), 1)
    return liota & 63, (liota >> 6) & 63


def _shift_tap(h, dy, dx, col, row, fill=0.0):
    n = h.shape[0]
    s = 64 * dy + dx
    if s > 0:
        sh = jnp.concatenate([h[:, s:], jnp.zeros((n, s), h.dtype)], axis=1)
    elif s < 0:
        sh = jnp.concatenate([jnp.zeros((n, -s), h.dtype), h[:, :LW + s]],
                             axis=1)
    else:
        sh = h
    if dy == 0 and dx == 0:
        return sh
    m = (col >= -dx) & (col < 64 - dx) & (row >= -dy) & (row < 64 - dy)
    return jnp.where(m, sh, jnp.asarray(fill, h.dtype))


def _nb(nsel, segb):
    return lax.dot_general(nsel, segb, (((1,), (0,)), ((), ())),
                           preferred_element_type=_F32)


def _bn2(h):
    m = jnp.mean(h, axis=1, keepdims=True)
    v = jnp.mean(h * h, axis=1, keepdims=True) - m * m
    return (h - m) * lax.rsqrt(v + 1e-5)


def _full_specs(shapes):
    return [pl.BlockSpec(s, lambda *a, _n=len(s): tuple(0 for _ in range(_n)))
            for s in shapes]


def _row_spec(cols):
    return pl.BlockSpec((RB, cols), lambda i: (i, 0))


def _cm(w):
    return lambda t: w[:, t].reshape(w.shape[0], 1)


def _dw1_body(xr_ref, nsel_ref, segb_ref, w3_ref, w5_ref, wd3_ref, wd5_ref,
              a3_ref, a5_ref, ad3_ref, ad5_ref):
    col, row = _geom()
    xt = xr_ref[...] * _nb(nsel_ref[...], segb_ref[...])
    rx = jax.nn.relu(xt).astype(_BF)
    cm = lambda r: _cm(r[...].astype(_BF))
    w3, w5, wd3, wd5 = cm(w3_ref), cm(w5_ref), cm(wd3_ref), cm(wd5_ref)
    z = jnp.zeros((RB, LW), _BF)
    a3, a5, ad3, ad5 = z, z, z, z
    for dy in range(-2, 3):
        for dx in range(-2, 3):
            t = _shift_tap(rx, dy, dx, col, row)
            a5 = a5 + w5((dy + 2) * 5 + (dx + 2)) * t
            if abs(dy) <= 1 and abs(dx) <= 1:
                a3 = a3 + w3((dy + 1) * 3 + (dx + 1)) * t
            if dy % 2 == 0 and dx % 2 == 0:
                ad3 = ad3 + wd3((dy // 2 + 1) * 3 + (dx // 2 + 1)) * t
                ad5 = ad5 + wd5((dy // 2 + 2) * 5 + (dx // 2 + 2)) * t
    for dy in (-4, -2, 0, 2, 4):
        for dx in (-4, -2, 0, 2, 4):
            if abs(dy) < 4 and abs(dx) < 4:
                continue
            t = _shift_tap(rx, dy, dx, col, row)
            ad5 = ad5 + wd5((dy // 2 + 2) * 5 + (dx // 2 + 2)) * t
    a3_ref[...] = a3
    a5_ref[...] = a5
    ad3_ref[...] = ad3
    ad5_ref[...] = ad5


def _dw1(xr2, nsel, segb, w3, w5, wd3, wd5):
    sh = jax.ShapeDtypeStruct((CC, LW), _BF)
    return pl.pallas_call(
        _dw1_body,
        grid=(NRB,),
        in_specs=[_row_spec(LW), _row_spec(B),
                  pl.BlockSpec((B, LW), lambda i: (0, 0)),
                  _row_spec(9), _row_spec(25), _row_spec(9), _row_spec(25)],
        out_specs=[_row_spec(LW)] * 4,
        out_shape=[sh] * 4,
    )(xr2, nsel, segb, w3, w5, wd3, wd5)


def _pool_body(xr_ref, nsel_ref, segb_ref, mp_ref, ap_ref):
    col, row = _geom()
    xb = (xr_ref[...] * _nb(nsel_ref[...], segb_ref[...])).astype(_BF)
    mp = None
    ap = jnp.zeros((RB, LW), _BF)
    for dy in (-1, 0, 1):
        for dx in (-1, 0, 1):
            mp_t = _shift_tap(xb, dy, dx, col, row, _NEG_INF)
            mp = mp_t if mp is None else jnp.maximum(mp, mp_t)
            ap = ap + _shift_tap(xb, dy, dx, col, row, 0.0)
    mp_ref[...] = _bn2(mp.astype(_F32)).astype(_BF)
    ap_ref[...] = _bn2(ap.astype(_F32) * (1.0 / 9.0)).astype(_BF)


def _pool(xr2, nsel, segb):
    sh = jax.ShapeDtypeStruct((CC, LW), _BF)
    return pl.pallas_call(
        _pool_body,
        grid=(NRB,),
        in_specs=[_row_spec(LW), _row_spec(B),
                  pl.BlockSpec((B, LW), lambda i: (0, 0))],
        out_specs=[_row_spec(LW)] * 2,
        out_shape=[sh, sh],
    )(xr2, nsel, segb)


def _pwbn(accs, ws, with_relu):
    n = len(accs)

    def body(*refs):
        in_refs, out_refs = refs[:2 * n], refs[2 * n:]
        for i in range(n):
            h = lax.dot_general(in_refs[n + i][...].astype(_BF),
                                in_refs[i][...], (((1,), (0,)), ((), ())),
                                preferred_element_type=_F32)
            h = _bn2(h)
            if with_relu:
                h = jax.nn.relu(h)
            out_refs[i][...] = h.astype(_BF)

    sh = jax.ShapeDtypeStruct((CC, LW), _BF)
    return pl.pallas_call(
        body,
        in_specs=_full_specs([(CC, LW)] * n + [(CC, CC)] * n),
        out_specs=_full_specs([(CC, LW)] * n),
        out_shape=[sh] * n,
    )(*accs, *ws)


def _dw2_body(h3_ref, h5_ref, w3_ref, w5_ref, a3_ref, a5_ref):
    col, row = _geom()

    def one(h_ref, w_ref, o_ref, k):
        p = k // 2
        hh = h_ref[...]
        wc = _cm(w_ref[...].astype(_BF))
        a = jnp.zeros((RB, LW), _BF)
        for dy in range(-p, p + 1):
            for dx in range(-p, p + 1):
                t = _shift_tap(hh, dy, dx, col, row)
                a = a + wc((dy + p) * k + (dx + p)) * t
        o_ref[...] = a

    one(h3_ref, w3_ref, a3_ref, 3)
    one(h5_ref, w5_ref, a5_ref, 5)


def _dw2(h3, h5, w3b, w5b):
    sh = jax.ShapeDtypeStruct((CC, LW), _BF)
    return pl.pallas_call(
        _dw2_body,
        grid=(NRB,),
        in_specs=[_row_spec(LW), _row_spec(LW), _row_spec(9), _row_spec(25)],
        out_specs=[_row_spec(LW)] * 2,
        out_shape=[sh, sh],
    )(h3, h5, w3b, w5b)


def _mean_body(xr_ref, nsel_ref, segp_ref, mp_ref, ap_ref, s3_ref, s5_ref,
               d3_ref, d5_ref, aw1_ref, aw2_ref, z_ref, oat_ref):
    segp = segp_ref[...].astype(_BF)
    segm = lambda r: lax.dot_general(r[...], segp, (((1,), (0,)), ((), ())),
                                     preferred_element_type=_F32)
    xt_sm = segm(xr_ref) * nsel_ref[...]
    y = jnp.concatenate([jnp.zeros((CC, B), _F32), segm(mp_ref),
                         segm(ap_ref), xt_sm, segm(s3_ref), segm(s5_ref),
                         segm(d3_ref), segm(d5_ref)], axis=0)
    h1 = jax.nn.relu(
        lax.dot_general(aw1_ref[...], y, (((1,), (0,)), ((), ())),
                        preferred_element_type=_F32))
    z = jax.nn.sigmoid(
        lax.dot_general(aw2_ref[...], h1, (((1,), (0,)), ((), ())),
                        preferred_element_type=_F32))
    z_ref[...] = z
    oat = [jnp.sum(z[i * CC:(i + 1) * CC]).reshape(1, 1) for i in range(8)]
    oat_ref[...] = jnp.concatenate(oat, axis=1)


def _mean(xr2b, nsel, segp, mp, ap, s3, s5, d3, d5, aw1, aw2):
    return pl.pallas_call(
        _mean_body,
        in_specs=_full_specs([(CC, LW), (CC, B), (LW, B)] + [(CC, LW)] * 6
                             + [(CC, AC), (AC, CC)]),
        out_specs=_full_specs([(AC, B), (1, 8)]),
        out_shape=[jax.ShapeDtypeStruct((AC, B), _F32),
                   jax.ShapeDtypeStruct((1, 8), _F32)],
    )(xr2b, nsel, segp, mp, ap, s3, s5, d3, d5, aw1, aw2)


def _acc_body(xr_ref, nsel_ref, segb_ref, zr_ref, mp_ref, ap_ref, s3_ref,
              s5_ref, d3_ref, d5_ref, out_ref):
    segb = segb_ref[...]
    zr = zr_ref[...]
    zb = lambda i: _nb(zr[:, i, :], segb)
    xt = xr_ref[...] * _nb(nsel_ref[...], segb)
    acc = zb(3) * xt
    acc = acc + zb(1) * mp_ref[...].astype(_F32)
    acc = acc + zb(2) * ap_ref[...].astype(_F32)
    acc = acc + zb(4) * s3_ref[...].astype(_F32)
    acc = acc + zb(5) * s5_ref[...].astype(_F32)
    acc = acc + zb(6) * d3_ref[...].astype(_F32)
    acc = acc + zb(7) * d5_ref[...].astype(_F32)
    out_ref[...] = acc


def _acc(xr2, nsel, segb, zr, mp, ap, s3, s5, d3, d5):
    return pl.pallas_call(
        _acc_body,
        grid=(NRB,),
        in_specs=[_row_spec(LW), _row_spec(B),
                  pl.BlockSpec((B, LW), lambda i: (0, 0)),
                  pl.BlockSpec((RB, 8, B), lambda i: (i, 0, 0))]
                 + [_row_spec(LW)] * 6,
        out_specs=_row_spec(LW),
        out_shape=jax.ShapeDtypeStruct((CC, LW), _F32),
    )(xr2, nsel, segb, zr, mp, ap, s3, s5, d3, d5)


def _assemble_body(idx_ref, x_ref, s_ref, conv_ref, o_ref):
    i = pl.program_id(0)
    base = i * CB
    o_ref[...] = x_ref[...] * s_ref[0][:, :, None]

    def body(j, _):
        c = idx_ref[j]
        lc = jnp.clip(c - base, 0, CB - 1)

        @pl.when((c >= base) & (c < base + CB))
        def _():
            o_ref[:, pl.ds(lc, 1), :] = (x_ref[:, pl.ds(lc, 1), :]
                                         + conv_ref[:, pl.ds(j, 1), :])
        return 0

    lax.fori_loop(0, CC, body, 0, unroll=False)


def _assemble(idx, x3, s, conv_b3):
    s3d = s.reshape(B, NCB, CB).transpose(1, 0, 2)
    grid_spec = pltpu.PrefetchScalarGridSpec(
        num_scalar_prefetch=1,
        grid=(NCB,),
        in_specs=[
            pl.BlockSpec((B, CB, HW), lambda i, idx_ref: (0, i, 0)),
            pl.BlockSpec((1, B, CB), lambda i, idx_ref: (i, 0, 0)),
            pl.BlockSpec((B, CC, HW), lambda i, idx_ref: (0, 0, 0)),
        ],
        out_specs=pl.BlockSpec((B, CB, HW), lambda i, idx_ref: (0, i, 0)),
    )
    return pl.pallas_call(
        _assemble_body,
        grid_spec=grid_spec,
        out_shape=jax.ShapeDtypeStruct((B, C, HW), _F32),
    )(idx, x3, s3d, conv_b3)


def kernel(x, ca_w1, ca_w2, sep3_dw1, sep3_pw1, sep3_dw2, sep3_pw2,
           sep5_dw1, sep5_pw1, sep5_dw2, sep5_pw2, dil3_dw, dil3_pw,
           dil5_dw, dil5_pw, att_w1, att_w2):
    x3 = x.reshape(B, C, HW)
    mx, av = _reduce(x3)
    num, s, idx2, nsel = _se_topk(mx, av, ca_w1, ca_w2)
    idx = idx2.reshape(CC)
    xr2 = _gather(idx, x).reshape(CC, LW)

    seg = jnp.arange(LW, dtype=jnp.int32) // HW
    segp = (seg[:, None] == jnp.arange(B)[None, :]).astype(_F32) / HW
    segb = (jnp.arange(B)[:, None] == seg[None, :]).astype(_F32)

    a3, a5, ad3, ad5 = _dw1(
        xr2, nsel, segb,
        sep3_dw1.reshape(CC, 9), sep5_dw1.reshape(CC, 25),
        dil3_dw.reshape(CC, 9), dil5_dw.reshape(CC, 25))
    mp, ap = _pool(xr2, nsel, segb)
    h3, h5 = _pwbn([a3, a5], [sep3_pw1.reshape(CC, CC),
                              sep5_pw1.reshape(CC, CC)], True)
    a3b, a5b = _dw2(h3, h5, sep3_dw2.reshape(CC, 9), sep5_dw2.reshape(CC, 25))
    s3, s5, d3, d5 = _pwbn(
        [a3b, a5b, ad3, ad5],
        [sep3_pw2.reshape(CC, CC), sep5_pw2.reshape(CC, CC),
         dil3_pw.reshape(CC, CC), dil5_pw.reshape(CC, CC)], False)

    xr2b = xr2.astype(_BF)
    z, oat = _mean(xr2b, nsel, segp, mp, ap, s3, s5, d3, d5, att_w1, att_w2)
    zr = z.reshape(8, CC, B).transpose(1, 0, 2)
    out2d = _acc(xr2, nsel, segb, zr, mp, ap, s3, s5, d3, d5)

    conv_b3 = out2d.reshape(CC, B, HW).transpose(1, 0, 2)
    xg = _assemble(idx, x3, s, conv_b3).reshape(B, C, H, W)
    return xg, oat.reshape(8)


# speedup vs baseline: 1.0614x; 1.0614x over previous
import jax
import jax.numpy as jnp
from jax import lax
from jax.experimental import pallas as pl
from jax.experimental.pallas import tpu as pltpu

B, C, H, W = 8, 768, 64, 64
K = 16
CC = C // K
AC = CC * 8
HW = H * W
LW = B * HW
CB = 64
NCB = C // CB
RB = 8
NRB = CC // RB
_NEG_INF = float("-inf")
_BF = jnp.bfloat16
_F32 = jnp.float32


def _reduce_body(x_ref, mx_ref, av_ref):
    xb = x_ref[...]
    mx_ref[...] = jnp.max(xb, axis=2)
    av_ref[...] = jnp.sum(xb, axis=2) * (1.0 / HW)


def _reduce(x3):
    return pl.pallas_call(
        _reduce_body,
        grid=(C // 128,),
        in_specs=[pl.BlockSpec((B, 128, HW), lambda i: (0, i, 0))],
        out_specs=[pl.BlockSpec((B, 128), lambda i: (0, i)),
                   pl.BlockSpec((B, 128), lambda i: (0, i))],
        out_shape=[jax.ShapeDtypeStruct((B, C), _F32),
                   jax.ShapeDtypeStruct((B, C), _F32)],
    )(x3)


def _se_topk_body(mx_ref, av_ref, w1_ref, w2_ref, s_ref, idx_ref, nsel_ref):
    v = jnp.concatenate([mx_ref[...], av_ref[...]], axis=0)
    h = jax.nn.relu(
        lax.dot_general(v, w1_ref[...], (((1,), (1,)), ((), ())),
                        preferred_element_type=_F32))
    r = lax.dot_general(h, w2_ref[...], (((1,), (1,)), ((), ())),
                        preferred_element_type=_F32)
    num = jax.nn.sigmoid(r[:B] + r[B:])
    slist = jnp.sum(num, axis=0, keepdims=True)
    iota = lax.broadcasted_iota(jnp.int32, (1, C), 1)
    iota48 = lax.broadcasted_iota(jnp.int32, (1, CC), 1)

    def body(i, carry):
        cur, mask, idxv = carry
        m = jnp.max(cur)
        j = jnp.min(jnp.where(cur == m, iota, C))
        idxv = jnp.where(iota48 == i, j, idxv)
        hit = iota == j
        return (jnp.where(hit, _NEG_INF, cur),
                jnp.where(hit, 1.0, mask), idxv)

    _, mask, idxv = lax.fori_loop(
        0, CC, body,
        (slist, jnp.zeros((1, C), _F32), jnp.zeros((1, CC), jnp.int32)),
        unroll=False)
    idx_ref[...] = idxv
    s = jnp.where(mask > 0.5, 1.0, 1.0 + num)
    for n in range(NCB):
        s_ref[n] = s[:, n * CB:(n + 1) * CB]
    oh = (lax.broadcasted_iota(jnp.int32, (CC, C), 1)
          == idxv.reshape(CC, 1)).astype(_F32)
    nsel_ref[...] = lax.dot_general(oh, num, (((1,), (1,)), ((), ())),
                                    preferred_element_type=_F32)


def _se_topk(mx, av, ca_w1, ca_w2):
    return pl.pallas_call(
        _se_topk_body,
        in_specs=[
            pl.BlockSpec((B, C), lambda: (0, 0)),
            pl.BlockSpec((B, C), lambda: (0, 0)),
            pl.BlockSpec((C // 2, C), lambda: (0, 0)),
            pl.BlockSpec((C, C // 2), lambda: (0, 0)),
        ],
        out_specs=[pl.BlockSpec((NCB, B, CB), lambda: (0, 0, 0)),
                   pl.BlockSpec((1, CC), lambda: (0, 0)),
                   pl.BlockSpec((CC, B), lambda: (0, 0))],
        out_shape=[jax.ShapeDtypeStruct((NCB, B, CB), _F32),
                   jax.ShapeDtypeStruct((1, CC), jnp.int32),
                   jax.ShapeDtypeStruct((CC, B), _F32)],
    )(mx, av, ca_w1, ca_w2)


def _gather_body(idx_ref, x_ref, out_ref):
    out_ref[...] = x_ref[...].reshape(1, B, H, W)


def _gather(idx, x):
    grid_spec = pltpu.PrefetchScalarGridSpec(
        num_scalar_prefetch=1,
        grid=(CC,),
        in_specs=[
            pl.BlockSpec((B, 1, H, W), lambda j, idx_ref: (0, idx_ref[j], 0, 0)),
        ],
        out_specs=pl.BlockSpec((1, B, H, W), lambda j, idx_ref: (j, 0, 0, 0)),
    )
    return pl.pallas_call(
        _gather_body,
        grid_spec=grid_spec,
        out_shape=jax.ShapeDtypeStruct((CC, B, H, W), _F32),
    )(idx, x)


def _geom():
    liota = lax.broadcasted_iota(jnp.int32, (1, LW), 1)
    return liota & 63, (liota >> 6) & 63


def _shift_tap(h, dy, dx, col, row, fill=0.0):
    n = h.shape[0]
    s = 64 * dy + dx
    if s > 0:
        sh = jnp.concatenate([h[:, s:], jnp.zeros((n, s), h.dtype)], axis=1)
    elif s < 0:
        sh = jnp.concatenate([jnp.zeros((n, -s), h.dtype), h[:, :LW + s]],
                             axis=1)
    else:
        sh = h
    if dy == 0 and dx == 0:
        return sh
    m = (col >= -dx) & (col < 64 - dx) & (row >= -dy) & (row < 64 - dy)
    return jnp.where(m, sh, jnp.asarray(fill, h.dtype))


def _nb(nsel, segb):
    return lax.dot_general(nsel, segb, (((1,), (0,)), ((), ())),
                           preferred_element_type=_F32)


def _bn2(h):
    m = jnp.mean(h, axis=1, keepdims=True)
    v = jnp.mean(h * h, axis=1, keepdims=True) - m * m
    return (h - m) * lax.rsqrt(v + 1e-5)


def _full_specs(shapes):
    return [pl.BlockSpec(s, lambda *a, _n=len(s): tuple(0 for _ in range(_n)))
            for s in shapes]


def _row_spec(cols):
    return pl.BlockSpec((RB, cols), lambda i: (i, 0))


def _cm(w):
    return lambda t: w[:, t].reshape(w.shape[0], 1)


def _dw1_body(xr_ref, nsel_ref, segb_ref, w3_ref, w5_ref, wd3_ref, wd5_ref,
              a3_ref, a5_ref, ad3_ref, ad5_ref):
    col, row = _geom()
    rx = jax.nn.relu(xr_ref[...] * _nb(nsel_ref[...], segb_ref[...]))
    w3, w5, wd3, wd5 = (_cm(w3_ref[...]), _cm(w5_ref[...]),
                        _cm(wd3_ref[...]), _cm(wd5_ref[...]))
    z = jnp.zeros((RB, LW), _F32)
    a3, a5, ad3, ad5 = z, z, z, z
    for dy in range(-2, 3):
        for dx in range(-2, 3):
            t = _shift_tap(rx, dy, dx, col, row)
            a5 = a5 + w5((dy + 2) * 5 + (dx + 2)) * t
            if abs(dy) <= 1 and abs(dx) <= 1:
                a3 = a3 + w3((dy + 1) * 3 + (dx + 1)) * t
            if dy % 2 == 0 and dx % 2 == 0:
                ad3 = ad3 + wd3((dy // 2 + 1) * 3 + (dx // 2 + 1)) * t
                ad5 = ad5 + wd5((dy // 2 + 2) * 5 + (dx // 2 + 2)) * t
    for dy in (-4, -2, 0, 2, 4):
        for dx in (-4, -2, 0, 2, 4):
            if abs(dy) < 4 and abs(dx) < 4:
                continue
            t = _shift_tap(rx, dy, dx, col, row)
            ad5 = ad5 + wd5((dy // 2 + 2) * 5 + (dx // 2 + 2)) * t
    a3_ref[...] = a3.astype(_BF)
    a5_ref[...] = a5.astype(_BF)
    ad3_ref[...] = ad3.astype(_BF)
    ad5_ref[...] = ad5.astype(_BF)


def _dw1(xr2, nsel, segb, w3, w5, wd3, wd5):
    sh = jax.ShapeDtypeStruct((CC, LW), _BF)
    return pl.pallas_call(
        _dw1_body,
        grid=(NRB,),
        in_specs=[_row_spec(LW), _row_spec(B),
                  pl.BlockSpec((B, LW), lambda i: (0, 0)),
                  _row_spec(9), _row_spec(25), _row_spec(9), _row_spec(25)],
        out_specs=[_row_spec(LW)] * 4,
        out_shape=[sh] * 4,
    )(xr2, nsel, segb, w3, w5, wd3, wd5)


def _pool_body(xr_ref, nsel_ref, segb_ref, mp_ref, ap_ref):
    col, row = _geom()
    xb = xr_ref[...] * _nb(nsel_ref[...], segb_ref[...])
    mp = None
    ap = jnp.zeros((RB, LW), _F32)
    for dy in (-1, 0, 1):
        for dx in (-1, 0, 1):
            mp_t = _shift_tap(xb, dy, dx, col, row, _NEG_INF)
            mp = mp_t if mp is None else jnp.maximum(mp, mp_t)
            ap = ap + _shift_tap(xb, dy, dx, col, row, 0.0)
    mp_ref[...] = _bn2(mp).astype(_BF)
    ap_ref[...] = _bn2(ap * (1.0 / 9.0)).astype(_BF)


def _pool(xr2, nsel, segb):
    sh = jax.ShapeDtypeStruct((CC, LW), _BF)
    return pl.pallas_call(
        _pool_body,
        grid=(NRB,),
        in_specs=[_row_spec(LW), _row_spec(B),
                  pl.BlockSpec((B, LW), lambda i: (0, 0))],
        out_specs=[_row_spec(LW)] * 2,
        out_shape=[sh, sh],
    )(xr2, nsel, segb)


def _pwbn(accs, ws, with_relu):
    n = len(accs)

    def body(*refs):
        in_refs, out_refs = refs[:2 * n], refs[2 * n:]
        for i in range(n):
            h = lax.dot_general(in_refs[n + i][...].astype(_BF),
                                in_refs[i][...], (((1,), (0,)), ((), ())),
                                preferred_element_type=_F32)
            h = _bn2(h)
            if with_relu:
                h = jax.nn.relu(h)
            out_refs[i][...] = h.astype(_BF)

    sh = jax.ShapeDtypeStruct((CC, LW), _BF)
    return pl.pallas_call(
        body,
        in_specs=_full_specs([(CC, LW)] * n + [(CC, CC)] * n),
        out_specs=_full_specs([(CC, LW)] * n),
        out_shape=[sh] * n,
    )(*accs, *ws)


def _dw2_body(h3_ref, h5_ref, w3_ref, w5_ref, a3_ref, a5_ref):
    col, row = _geom()

    def one(h_ref, w_ref, o_ref, k):
        p = k // 2
        hh = h_ref[...].astype(_F32)
        wc = _cm(w_ref[...])
        a = jnp.zeros((RB, LW), _F32)
        for dy in range(-p, p + 1):
            for dx in range(-p, p + 1):
                t = _shift_tap(hh, dy, dx, col, row)
                a = a + wc((dy + p) * k + (dx + p)) * t
        o_ref[...] = a.astype(_BF)

    one(h3_ref, w3_ref, a3_ref, 3)
    one(h5_ref, w5_ref, a5_ref, 5)


def _dw2(h3, h5, w3b, w5b):
    sh = jax.ShapeDtypeStruct((CC, LW), _BF)
    return pl.pallas_call(
        _dw2_body,
        grid=(NRB,),
        in_specs=[_row_spec(LW), _row_spec(LW), _row_spec(9), _row_spec(25)],
        out_specs=[_row_spec(LW)] * 2,
        out_shape=[sh, sh],
    )(h3, h5, w3b, w5b)


def _mean_body(xr_ref, nsel_ref, segp_ref, mp_ref, ap_ref, s3_ref, s5_ref,
               d3_ref, d5_ref, aw1_ref, aw2_ref, z_ref, oat_ref):
    segp = segp_ref[...].astype(_BF)
    segm = lambda v: lax.dot_general(v, segp, (((1,), (0,)), ((), ())),
                                     preferred_element_type=_F32)
    xt_sm = segm(xr_ref[...].astype(_BF)) * nsel_ref[...]
    y = jnp.concatenate([jnp.zeros((CC, B), _F32), segm(mp_ref[...]),
                         segm(ap_ref[...]), xt_sm, segm(s3_ref[...]),
                         segm(s5_ref[...]), segm(d3_ref[...]),
                         segm(d5_ref[...])], axis=0)
    h1 = jax.nn.relu(
        lax.dot_general(aw1_ref[...], y, (((1,), (0,)), ((), ())),
                        preferred_element_type=_F32))
    z = jax.nn.sigmoid(
        lax.dot_general(aw2_ref[...], h1, (((1,), (0,)), ((), ())),
                        preferred_element_type=_F32))
    z_ref[...] = z
    oat = [jnp.sum(z[i * CC:(i + 1) * CC]).reshape(1, 1) for i in range(8)]
    oat_ref[...] = jnp.concatenate(oat, axis=1)


def _mean(xr2b, nsel, segp, mp, ap, s3, s5, d3, d5, aw1, aw2):
    return pl.pallas_call(
        _mean_body,
        in_specs=_full_specs([(CC, LW), (CC, B), (LW, B)] + [(CC, LW)] * 6
                             + [(CC, AC), (AC, CC)]),
        out_specs=_full_specs([(AC, B), (1, 8)]),
        out_shape=[jax.ShapeDtypeStruct((AC, B), _F32),
                   jax.ShapeDtypeStruct((1, 8), _F32)],
    )(xr2b, nsel, segp, mp, ap, s3, s5, d3, d5, aw1, aw2)


def _acc_body(xr_ref, nsel_ref, segb_ref, zr_ref, mp_ref, ap_ref, s3_ref,
              s5_ref, d3_ref, d5_ref, out_ref):
    segb = segb_ref[...]
    zr = zr_ref[...]
    zb = lambda i: _nb(zr[:, i, :], segb)
    xt = xr_ref[...] * _nb(nsel_ref[...], segb)
    acc = zb(3) * xt
    acc = acc + zb(1) * mp_ref[...].astype(_F32)
    acc = acc + zb(2) * ap_ref[...].astype(_F32)
    acc = acc + zb(4) * s3_ref[...].astype(_F32)
    acc = acc + zb(5) * s5_ref[...].astype(_F32)
    acc = acc + zb(6) * d3_ref[...].astype(_F32)
    acc = acc + zb(7) * d5_ref[...].astype(_F32)
    out_ref[...] = jnp.transpose(acc.reshape(RB, B, HW), (1, 0, 2))


def _acc(xr2, nsel, segb, zr, mp, ap, s3, s5, d3, d5):
    return pl.pallas_call(
        _acc_body,
        grid=(NRB,),
        in_specs=[_row_spec(LW), _row_spec(B),
                  pl.BlockSpec((B, LW), lambda i: (0, 0)),
                  pl.BlockSpec((RB, 8, B), lambda i: (i, 0, 0))]
                 + [_row_spec(LW)] * 6,
        out_specs=pl.BlockSpec((B, RB, HW), lambda i: (0, i, 0)),
        out_shape=jax.ShapeDtypeStruct((B, CC, HW), _F32),
    )(xr2, nsel, segb, zr, mp, ap, s3, s5, d3, d5)


def _assemble_body(idx_ref, x_ref, s_ref, conv_ref, o_ref):
    i = pl.program_id(0)
    base = i * CB
    o_ref[...] = x_ref[...] * s_ref[0][:, :, None]

    def body(j, _):
        c = idx_ref[j]
        lc = jnp.clip(c - base, 0, CB - 1)

        @pl.when((c >= base) & (c < base + CB))
        def _():
            o_ref[:, pl.ds(lc, 1), :] = (x_ref[:, pl.ds(lc, 1), :]
                                         + conv_ref[:, pl.ds(j, 1), :])
        return 0

    lax.fori_loop(0, CC, body, 0, unroll=False)


def _assemble(idx, x3, s3d, conv_b3):
    grid_spec = pltpu.PrefetchScalarGridSpec(
        num_scalar_prefetch=1,
        grid=(NCB,),
        in_specs=[
            pl.BlockSpec((B, CB, HW), lambda i, idx_ref: (0, i, 0)),
            pl.BlockSpec((1, B, CB), lambda i, idx_ref: (i, 0, 0)),
            pl.BlockSpec((B, CC, HW), lambda i, idx_ref: (0, 0, 0)),
        ],
        out_specs=pl.BlockSpec((B, CB, HW), lambda i, idx_ref: (0, i, 0)),
    )
    return pl.pallas_call(
        _assemble_body,
        grid_spec=grid_spec,
        out_shape=jax.ShapeDtypeStruct((B, C, HW), _F32),
    )(idx, x3, s3d, conv_b3)


def kernel(x, ca_w1, ca_w2, sep3_dw1, sep3_pw1, sep3_dw2, sep3_pw2,
           sep5_dw1, sep5_pw1, sep5_dw2, sep5_pw2, dil3_dw, dil3_pw,
           dil5_dw, dil5_pw, att_w1, att_w2):
    x3 = x.reshape(B, C, HW)
    mx, av = _reduce(x3)
    s3d, idx2, nsel = _se_topk(mx, av, ca_w1, ca_w2)
    idx = idx2.reshape(CC)
    xr2 = _gather(idx, x).reshape(CC, LW)

    seg = jnp.arange(LW, dtype=jnp.int32) // HW
    segp = (seg[:, None] == jnp.arange(B)[None, :]).astype(_F32) / HW
    segb = (jnp.arange(B)[:, None] == seg[None, :]).astype(_F32)

    a3, a5, ad3, ad5 = _dw1(
        xr2, nsel, segb,
        sep3_dw1.reshape(CC, 9), sep5_dw1.reshape(CC, 25),
        dil3_dw.reshape(CC, 9), dil5_dw.reshape(CC, 25))
    mp, ap = _pool(xr2, nsel, segb)
    h3, h5 = _pwbn([a3, a5], [sep3_pw1.reshape(CC, CC),
                              sep5_pw1.reshape(CC, CC)], True)
    a3b, a5b = _dw2(h3, h5, sep3_dw2.reshape(CC, 9), sep5_dw2.reshape(CC, 25))
    s3, s5, d3, d5 = _pwbn(
        [a3b, a5b, ad3, ad5],
        [sep3_pw2.reshape(CC, CC), sep5_pw2.reshape(CC, CC),
         dil3_pw.reshape(CC, CC), dil5_pw.reshape(CC, CC)], False)

    z, oat = _mean(xr2, nsel, segp, mp, ap, s3, s5, d3, d5, att_w1, att_w2)
    zr = z.reshape(8, CC, B).transpose(1, 0, 2)
    conv_b3 = _acc(xr2, nsel, segb, zr, mp, ap, s3, s5, d3, d5)
    xg = _assemble(idx, x3, s3d, conv_b3).reshape(B, C, H, W)
    return xg, oat.reshape(8)
